# baseline (speedup 1.0000x reference)
import functools

import jax
import jax.numpy as jnp
from jax.experimental import pallas as pl
from jax.experimental.pallas import tpu as pltpu

N = 4096
SIGMA = 0.2
EPSILON = 1.0
CUTOFF = 0.5
BR = 256


def _lj_block_kernel(pos_blk_ref, pos_all_ref, energy_ref, forces_ref):
    step = pl.program_id(0)

    pos_blk = pos_blk_ref[...]
    pos_all = pos_all_ref[...]

    dx = pos_all[0:1, :] - pos_blk[:, 0:1]
    dy = pos_all[1:2, :] - pos_blk[:, 1:2]
    dz = pos_all[2:3, :] - pos_blk[:, 2:3]
    d2 = dx * dx + dy * dy + dz * dz
    r = jnp.sqrt(d2)

    rows = jax.lax.broadcasted_iota(jnp.int32, (BR, N), 0) + step * BR
    cols = jax.lax.broadcasted_iota(jnp.int32, (BR, N), 1)
    valid = (r < CUTOFF) & (r > 0.0) & (rows != cols)

    safe_r = jnp.where(valid, r, 1.0)
    idr = SIGMA / safe_r
    idr2 = idr * idr
    idr6 = idr2 * idr2 * idr2
    idr12 = idr6 * idr6

    pair_e = jnp.where(valid, 4.0 * EPSILON * (idr12 - idr6), 0.0)
    e_part = 0.5 * jnp.sum(pair_e)

    @pl.when(step == 0)
    def _():
        energy_ref[0, 0] = 0.0

    energy_ref[0, 0] += e_part

    pf = 24.0 * EPSILON / safe_r * (2.0 * idr12 - idr6)
    g = jnp.where(valid, pf / safe_r, 0.0)
    fx = 2.0 * jnp.sum(g * dx, axis=1, keepdims=True)
    fy = 2.0 * jnp.sum(g * dy, axis=1, keepdims=True)
    fz = 2.0 * jnp.sum(g * dz, axis=1, keepdims=True)
    forces_ref[...] = jnp.concatenate([fx, fy, fz], axis=1)


@jax.jit
def kernel(positions, cell):
    del cell
    pos_t = positions.T
    grid = (N // BR,)
    energy, forces = pl.pallas_call(
        _lj_block_kernel,
        grid=grid,
        in_specs=[
            pl.BlockSpec((BR, 3), lambda i: (i, 0)),
            pl.BlockSpec((3, N), lambda i: (0, 0)),
        ],
        out_specs=[
            pl.BlockSpec(memory_space=pltpu.SMEM, block_shape=(1, 1), index_map=lambda i: (0, 0)),
            pl.BlockSpec((BR, 3), lambda i: (i, 0)),
        ],
        out_shape=[
            jax.ShapeDtypeStruct((1, 1), jnp.float32),
            jax.ShapeDtypeStruct((N, 3), jnp.float32),
        ],
    )(positions, pos_t)
    return energy[0, 0], forces


# speedup vs baseline: 1.0732x; 1.0732x over previous
import functools

import jax
import jax.numpy as jnp
from jax.experimental import pallas as pl
from jax.experimental.pallas import tpu as pltpu

N = 4096
SIGMA = 0.2
EPSILON = 1.0
CUTOFF = 0.5
BR = 512


def _lj_block_kernel(pos_blk_ref, pos_all_ref, energy_ref, forces_ref):
    step = pl.program_id(0)

    pos_blk = pos_blk_ref[...]
    pos_all = pos_all_ref[...]

    dx = pos_all[0:1, :] - pos_blk[:, 0:1]
    dy = pos_all[1:2, :] - pos_blk[:, 1:2]
    dz = pos_all[2:3, :] - pos_blk[:, 2:3]
    d2 = dx * dx + dy * dy + dz * dz
    r = jnp.sqrt(d2)

    valid = (r < CUTOFF) & (r > 0.0)

    idr = SIGMA / r
    idr2 = idr * idr
    idr6 = idr2 * idr2 * idr2
    idr12 = idr6 * idr6

    pair_e = jnp.where(valid, 4.0 * EPSILON * (idr12 - idr6), 0.0)
    e_part = 0.5 * jnp.sum(pair_e)

    @pl.when(step == 0)
    def _():
        energy_ref[0, 0] = 0.0

    energy_ref[0, 0] += e_part

    pf = 24.0 * EPSILON / r * (2.0 * idr12 - idr6)
    g = jnp.where(valid, pf / r, 0.0)
    fx = 2.0 * jnp.sum(g * dx, axis=1, keepdims=True)
    fy = 2.0 * jnp.sum(g * dy, axis=1, keepdims=True)
    fz = 2.0 * jnp.sum(g * dz, axis=1, keepdims=True)
    forces_ref[...] = jnp.concatenate([fx, fy, fz], axis=1)


@jax.jit
def kernel(positions, cell):
    del cell
    pos_t = positions.T
    grid = (N // BR,)
    energy, forces = pl.pallas_call(
        _lj_block_kernel,
        grid=grid,
        in_specs=[
            pl.BlockSpec((BR, 3), lambda i: (i, 0)),
            pl.BlockSpec((3, N), lambda i: (0, 0)),
        ],
        out_specs=[
            pl.BlockSpec(memory_space=pltpu.SMEM, block_shape=(1, 1), index_map=lambda i: (0, 0)),
            pl.BlockSpec((BR, 3), lambda i: (i, 0)),
        ],
        out_shape=[
            jax.ShapeDtypeStruct((1, 1), jnp.float32),
            jax.ShapeDtypeStruct((N, 3), jnp.float32),
        ],
    )(positions, pos_t)
    return energy[0, 0], forces
